# baseline (speedup 1.0000x reference)
import functools

import jax
import jax.numpy as jnp
from jax import lax
from jax.experimental import pallas as pl
from jax.experimental.pallas import tpu as pltpu
from jax.experimental.pallas import tpu_sc as plsc

L = 256
D = 32768
NC = 2
NS = 16
NW = NC * NS
RPW = L // NW


def _permute_body(x_hbm, perm_hbm, out_hbm, pvm, sems):
    c = lax.axis_index("c")
    s = lax.axis_index("s")
    wid = s * NC + c
    base = wid * RPW

    pltpu.sync_copy(perm_hbm.at[pl.ds(base, RPW)], pvm.at[pl.ds(0, RPW)])
    vals = pvm[...]

    cps = []
    for k in range(RPW):
        srow = vals[k]
        cps.append(pltpu.async_copy(x_hbm.at[pl.ds(srow, 1)],
                                    out_hbm.at[pl.ds(base + k, 1)], sems[k]))
    for cp in cps:
        cp.wait()


@functools.partial(
    pl.kernel,
    out_type=jax.ShapeDtypeStruct((L, D), jnp.float32),
    mesh=plsc.VectorSubcoreMesh(core_axis_name="c", subcore_axis_name="s"),
    scratch_types=[
        pltpu.VMEM((16,), jnp.int32),
        [pltpu.SemaphoreType.DMA] * RPW,
    ],
)
def _permute(x_hbm, perm_hbm, out_hbm, pvm, sems):
    _permute_body(x_hbm, perm_hbm, out_hbm, pvm, sems)


def kernel(x, permutations):
    perm1d = permutations.astype(jnp.int32)
    return _permute(x, perm1d)


# speedup vs baseline: 24.1389x; 24.1389x over previous
import functools

import jax
import jax.numpy as jnp
from jax import lax
from jax.experimental import pallas as pl
from jax.experimental.pallas import tpu as pltpu
from jax.experimental.pallas import tpu_sc as plsc

L = 256
D = 32768
NC = 2
NS = 16
NW = NC * NS
RPW = L // NW

NBUF = 3


def _permute_body(x_hbm, perm_hbm, out_hbm, pvm, bufs, gsems, ssems):
    c = lax.axis_index("c")
    s = lax.axis_index("s")
    wid = s * NC + c
    base = wid * RPW

    pltpu.sync_copy(perm_hbm.at[pl.ds(base, RPW)], pvm.at[pl.ds(0, RPW)])
    vals = pvm[...]

    g = [None] * RPW
    st = [None] * RPW
    for k in range(NBUF):
        g[k] = pltpu.async_copy(x_hbm.at[pl.ds(vals[k], 1)], bufs[k],
                                gsems[k])
    for k in range(RPW):
        sl = k % NBUF
        g[k].wait()
        st[k] = pltpu.async_copy(bufs[sl], out_hbm.at[pl.ds(base + k, 1)],
                                 ssems[sl])
        if k + NBUF < RPW:
            st[k].wait()
            g[k + NBUF] = pltpu.async_copy(
                x_hbm.at[pl.ds(vals[k + NBUF], 1)], bufs[sl], gsems[sl])
    for k in range(RPW - NBUF, RPW):
        if st[k] is not None:
            st[k].wait()


@functools.partial(
    pl.kernel,
    out_type=jax.ShapeDtypeStruct((L, D), jnp.float32),
    mesh=plsc.VectorSubcoreMesh(core_axis_name="c", subcore_axis_name="s"),
    scratch_types=[
        pltpu.VMEM((16,), jnp.int32),
        [pltpu.VMEM((1, D), jnp.float32)] * NBUF,
        [pltpu.SemaphoreType.DMA] * NBUF,
        [pltpu.SemaphoreType.DMA] * NBUF,
    ],
)
def _permute(x_hbm, perm_hbm, out_hbm, pvm, bufs, gsems, ssems):
    _permute_body(x_hbm, perm_hbm, out_hbm, pvm, bufs, gsems, ssems)


def kernel(x, permutations):
    perm1d = permutations.astype(jnp.int32)
    return _permute(x, perm1d)
